# baseline (speedup 1.0000x reference)
import functools

import jax
import jax.numpy as jnp
import numpy as np
from jax import lax
from jax.experimental import pallas as pl
from jax.experimental.pallas import tpu as pltpu
from jax.experimental.pallas import tpu_sc as plsc

VOCAB = 6400
EMB = 64
HALF = EMB // 2
SEQ = 20
KCH = SEQ // 4
BATCH = 16384
NHALVES = 2
BH = BATCH // NHALVES
NC = 2
NS = 16
NW = NC * NS
B_PER_W = BH // NW
ROWS_PER_W = B_PER_W * SEQ
CHUNK = B_PER_W * 4

_W = np.arange(128)
_PA = np.stack([(4 * k + _W // HALF) * EMB + _W % HALF for k in range(KCH)])
_PB = _PA + HALF


def _sc_gather(table_packed, idx, addrbase, hbase):
    mesh = plsc.VectorSubcoreMesh(core_axis_name="c", subcore_axis_name="s")

    @functools.partial(
        pl.kernel,
        mesh=mesh,
        out_type=jax.ShapeDtypeStruct((KCH, BH * 4, HALF), jnp.int32),
        scratch_types=[
            pltpu.VMEM((ROWS_PER_W,), jnp.int32),
            pltpu.VMEM((16,), jnp.int32),
            pltpu.VMEM((CHUNK,), jnp.int32),
            pltpu.VMEM((CHUNK,), jnp.int32),
            pltpu.VMEM((CHUNK, HALF), jnp.int32),
            pltpu.VMEM((CHUNK, HALF), jnp.int32),
            pltpu.VMEM_SHARED((VOCAB, HALF), jnp.int32),
            pltpu.SemaphoreType.DMA,
            pltpu.SemaphoreType.DMA,
            pltpu.SemaphoreType.DMA,
            pltpu.SemaphoreType.DMA,
        ],
        compiler_params=pltpu.CompilerParams(use_tc_tiling_on_sc=False,
                                             needs_layout_passes=False),
    )
    def k(table_hbm, idx_hbm, ab_hbm, out_hbm, idx_v, addr_v, ib0, ib1,
          buf0, buf1, spm_tab, g0, g1, w0, w1):
        wid = lax.axis_index("s") * NC + lax.axis_index("c")

        @pl.when(lax.axis_index("s") == 0)
        def _():
            pltpu.sync_copy(table_hbm, spm_tab)

        pltpu.sync_copy(idx_hbm.at[pl.ds(wid * ROWS_PER_W, ROWS_PER_W)],
                        idx_v)
        pltpu.sync_copy(ab_hbm, addr_v)
        plsc.subcore_barrier()

        def build(ib):
            def bb(j, carry):
                a = addr_v[...]
                ib[pl.ds(j * 16, 16)] = plsc.load_gather(idx_v, [a])
                addr_v[...] = a + 4 * SEQ
                return carry

            lax.fori_loop(0, CHUNK // 16, bb, 0)
            addr_v[...] = addr_v[...] - (ROWS_PER_W - 4)

        def gat(buf, sem, ib):
            return pltpu.async_copy(spm_tab.at[ib], buf, sem)

        def wr(kk, buf, sem):
            off = pl.multiple_of(wid * CHUNK, CHUNK)
            return pltpu.async_copy(buf, out_hbm.at[kk, pl.ds(off, CHUNK)],
                                    sem)

        ibs = (ib0, ib1)
        bufs = (buf0, buf1)
        gsem = (g0, g1)
        wsem = (w0, w1)

        build(ibs[0])
        ghs = {0: gat(bufs[0], gsem[0], ibs[0])}
        whs = {}
        for t in range(KCH):
            if t + 1 < KCH:
                build(ibs[(t + 1) % 2])
                if t >= 1:
                    whs[t - 1].wait()
                ghs[t + 1] = gat(bufs[(t + 1) % 2], gsem[(t + 1) % 2],
                                 ibs[(t + 1) % 2])
            ghs[t].wait()
            whs[t] = wr(t, bufs[t % 2], wsem[t % 2])
        whs[KCH - 2].wait()
        whs[KCH - 1].wait()

    return k(table_packed, idx, addrbase)


TB = 2048


def _mlp_body(x_ref, w1_ref, b1_ref, w2_ref, b2_ref, w3_ref, b3_ref,
              o_in_ref, o_ref):
    h = None
    for k in range(KCH):
        xi = x_ref[k]
        xa = lax.bitcast_convert_type(xi << 16, jnp.float32)
        xb = lax.bitcast_convert_type(xi & jnp.int32(-65536), jnp.float32)
        d = jnp.dot(xa.astype(jnp.bfloat16), w1_ref[k, :128],
                    preferred_element_type=jnp.float32)
        d += jnp.dot(xb.astype(jnp.bfloat16), w1_ref[k, 128:],
                     preferred_element_type=jnp.float32)
        h = d if h is None else h + d
    h = jnp.maximum(h + b1_ref[...], 0.0)
    h = jnp.dot(h, w2_ref[...], preferred_element_type=jnp.float32)
    h = jnp.maximum(h + b2_ref[...], 0.0)
    o = jnp.dot(h, w3_ref[...], preferred_element_type=jnp.float32)
    o_ref[...] = o + b3_ref[...]


def _mlp(x, W1ab, b1, W2, b2, W3, b3, o_prev, half):
    grid = (BH // TB,)
    base = half * (BH // TB)
    return pl.pallas_call(
        _mlp_body,
        grid=grid,
        in_specs=[
            pl.BlockSpec((KCH, TB, 128), lambda i: (0, i, 0)),
            pl.BlockSpec((KCH, 256, 128), lambda i: (0, 0, 0)),
            pl.BlockSpec((1, 128), lambda i: (0, 0)),
            pl.BlockSpec((128, 64), lambda i: (0, 0)),
            pl.BlockSpec((1, 64), lambda i: (0, 0)),
            pl.BlockSpec((64, 2), lambda i: (0, 0)),
            pl.BlockSpec((1, 2), lambda i: (0, 0)),
            pl.BlockSpec((TB, 2), lambda i: (i + base, 0)),
        ],
        out_specs=pl.BlockSpec((TB, 2), lambda i: (i + base, 0)),
        out_shape=jax.ShapeDtypeStruct((BATCH, 2), jnp.float32),
        input_output_aliases={7: 0},
    )(x, W1ab, b1, W2, b2, W3, b3, o_prev)


def kernel(inputs, table, W1, b1, W2, b2, W3, b3):
    tb = table.astype(jnp.bfloat16)
    lo = lax.bitcast_convert_type(tb[:, :HALF], jnp.uint16).astype(jnp.uint32)
    hi = lax.bitcast_convert_type(tb[:, HALF:], jnp.uint16).astype(jnp.uint32)
    packed = lax.bitcast_convert_type((hi << 16) | lo, jnp.int32)
    addrbase = jnp.asarray(
        [(l // 4) * SEQ + l % 4 for l in range(16)], dtype=jnp.int32)
    W1ab = jnp.concatenate([W1[_PA], W1[_PB]], axis=1)
    W1ab = W1ab.astype(jnp.bfloat16)
    b1r, b2r, b3r = b1.reshape(1, -1), b2.reshape(1, -1), b3.reshape(1, -1)

    out = jnp.zeros((BATCH, 2), jnp.float32)
    for h in range(NHALVES):
        idx_h = inputs[h * BH:(h + 1) * BH].astype(jnp.int32).reshape(-1)
        x = _sc_gather(packed, idx_h, addrbase, 0)
        x = x.reshape(KCH, BH, 128)
        out = _mlp(x, W1ab, b1r, W2, b2r, W3, b3r, out, h)
    return out


# speedup vs baseline: 1.0751x; 1.0751x over previous
import functools

import jax
import jax.numpy as jnp
import numpy as np
from jax import lax
from jax.experimental import pallas as pl
from jax.experimental.pallas import tpu as pltpu
from jax.experimental.pallas import tpu_sc as plsc

VOCAB = 6400
EMB = 64
HALF = EMB // 2
SEQ = 20
KCH = SEQ // 4
BATCH = 16384
NHALVES = 2
BH = BATCH // NHALVES
NC = 2
NS = 16
NW = NC * NS
B_PER_W = BH // NW
ROWS_PER_W = B_PER_W * SEQ
CHUNK = B_PER_W * 4

_W = np.arange(128)
_PA = np.stack([(4 * k + _W // HALF) * EMB + _W % HALF for k in range(KCH)])
_PB = _PA + HALF


def _sc_gather(table_packed, idx, addrbase, hbase):
    mesh = plsc.VectorSubcoreMesh(core_axis_name="c", subcore_axis_name="s")

    @functools.partial(
        pl.kernel,
        mesh=mesh,
        out_type=jax.ShapeDtypeStruct((KCH, BH * 4, HALF), jnp.int32),
        scratch_types=[
            pltpu.VMEM((ROWS_PER_W,), jnp.int32),
            pltpu.VMEM((16,), jnp.int32),
            pltpu.VMEM((CHUNK,), jnp.int32),
            pltpu.VMEM((CHUNK,), jnp.int32),
            pltpu.VMEM((CHUNK, HALF), jnp.int32),
            pltpu.VMEM((CHUNK, HALF), jnp.int32),
            pltpu.VMEM_SHARED((VOCAB, HALF), jnp.int32),
            pltpu.SemaphoreType.DMA,
            pltpu.SemaphoreType.DMA,
            pltpu.SemaphoreType.DMA,
            pltpu.SemaphoreType.DMA,
        ],
        compiler_params=pltpu.CompilerParams(use_tc_tiling_on_sc=False,
                                             needs_layout_passes=False),
    )
    def k(table_hbm, idx_hbm, ab_hbm, out_hbm, idx_v, addr_v, ib0, ib1,
          buf0, buf1, spm_tab, g0, g1, w0, w1):
        wid = lax.axis_index("s") * NC + lax.axis_index("c")

        @pl.when(lax.axis_index("s") == 0)
        def _():
            pltpu.sync_copy(table_hbm, spm_tab)

        pltpu.sync_copy(idx_hbm.at[pl.ds(wid * ROWS_PER_W, ROWS_PER_W)],
                        idx_v)
        pltpu.sync_copy(ab_hbm, addr_v)
        plsc.subcore_barrier()

        def build(ib):
            def bb(j, carry):
                a = addr_v[...]
                ib[pl.ds(j * 16, 16)] = plsc.load_gather(idx_v, [a])
                addr_v[...] = a + 4 * SEQ
                return carry

            lax.fori_loop(0, CHUNK // 16, bb, 0)
            addr_v[...] = addr_v[...] - (ROWS_PER_W - 4)

        def gat(buf, sem, ib):
            return pltpu.async_copy(spm_tab.at[ib], buf, sem)

        def wr(kk, buf, sem):
            off = pl.multiple_of(wid * CHUNK, CHUNK)
            return pltpu.async_copy(buf, out_hbm.at[kk, pl.ds(off, CHUNK)],
                                    sem)

        ibs = (ib0, ib1)
        bufs = (buf0, buf1)
        gsem = (g0, g1)
        wsem = (w0, w1)

        build(ibs[0])
        ghs = {0: gat(bufs[0], gsem[0], ibs[0])}
        whs = {}
        for t in range(KCH):
            if t + 1 < KCH:
                build(ibs[(t + 1) % 2])
                if t >= 1:
                    whs[t - 1].wait()
                ghs[t + 1] = gat(bufs[(t + 1) % 2], gsem[(t + 1) % 2],
                                 ibs[(t + 1) % 2])
            ghs[t].wait()
            whs[t] = wr(t, bufs[t % 2], wsem[t % 2])
        whs[KCH - 2].wait()
        whs[KCH - 1].wait()

    return k(table_packed, idx, addrbase)


TB = 2048


def _mlp_body(x_ref, w1_ref, b1_ref, w2_ref, b2_ref, w3_ref, b3_ref, o_ref):
    h = None
    for k in range(KCH):
        xi = x_ref[k]
        xa = lax.bitcast_convert_type(xi << 16, jnp.float32)
        xb = lax.bitcast_convert_type(xi & jnp.int32(-65536), jnp.float32)
        d = jnp.dot(xa.astype(jnp.bfloat16), w1_ref[k, :128],
                    preferred_element_type=jnp.float32)
        d += jnp.dot(xb.astype(jnp.bfloat16), w1_ref[k, 128:],
                     preferred_element_type=jnp.float32)
        h = d if h is None else h + d
    h = jnp.maximum(h + b1_ref[...], 0.0)
    h = jnp.dot(h, w2_ref[...], preferred_element_type=jnp.float32)
    h = jnp.maximum(h + b2_ref[...], 0.0)
    o = jnp.dot(h, w3_ref[...], preferred_element_type=jnp.float32)
    o_ref[...] = o + b3_ref[...]


def _mlp(x, W1ab, b1, W2, b2, W3, b3):
    grid = (BH // TB,)
    return pl.pallas_call(
        _mlp_body,
        grid=grid,
        in_specs=[
            pl.BlockSpec((KCH, TB, 128), lambda i: (0, i, 0)),
            pl.BlockSpec((KCH, 256, 128), lambda i: (0, 0, 0)),
            pl.BlockSpec((1, 128), lambda i: (0, 0)),
            pl.BlockSpec((128, 64), lambda i: (0, 0)),
            pl.BlockSpec((1, 64), lambda i: (0, 0)),
            pl.BlockSpec((64, 2), lambda i: (0, 0)),
            pl.BlockSpec((1, 2), lambda i: (0, 0)),
        ],
        out_specs=pl.BlockSpec((TB, 2), lambda i: (i, 0)),
        out_shape=jax.ShapeDtypeStruct((BH, 2), jnp.float32),
    )(x, W1ab, b1, W2, b2, W3, b3)


def kernel(inputs, table, W1, b1, W2, b2, W3, b3):
    tb = table.astype(jnp.bfloat16)
    lo = lax.bitcast_convert_type(tb[:, :HALF], jnp.uint16).astype(jnp.uint32)
    hi = lax.bitcast_convert_type(tb[:, HALF:], jnp.uint16).astype(jnp.uint32)
    packed = lax.bitcast_convert_type((hi << 16) | lo, jnp.int32)
    addrbase = jnp.asarray(
        [(l // 4) * SEQ + l % 4 for l in range(16)], dtype=jnp.int32)
    W1ab = jnp.concatenate([W1[_PA], W1[_PB]], axis=1)
    W1ab = W1ab.astype(jnp.bfloat16)
    b1r, b2r, b3r = b1.reshape(1, -1), b2.reshape(1, -1), b3.reshape(1, -1)

    outs = []
    for h in range(NHALVES):
        idx_h = inputs[h * BH:(h + 1) * BH].astype(jnp.int32).reshape(-1)
        x = _sc_gather(packed, idx_h, addrbase, 0)
        x = x.reshape(KCH, BH, 128)
        outs.append(_mlp(x, W1ab, b1r, W2, b2r, W3, b3r))
    return jnp.concatenate(outs, axis=0)


# speedup vs baseline: 1.0753x; 1.0002x over previous
import functools

import jax
import jax.numpy as jnp
import numpy as np
from jax import lax
from jax.experimental import pallas as pl
from jax.experimental.pallas import tpu as pltpu
from jax.experimental.pallas import tpu_sc as plsc

VOCAB = 6400
EMB = 64
HALF = EMB // 2
SEQ = 20
KCH = SEQ // 4
BATCH = 16384
NHALVES = 2
BH = BATCH // NHALVES
NC = 2
NS = 16
NW = NC * NS
B_PER_W = BH // NW
ROWS_PER_W = B_PER_W * SEQ
CHUNK = B_PER_W * 4

_W = np.arange(128)
_PA = np.stack([(4 * k + _W // HALF) * EMB + _W % HALF for k in range(KCH)])
_PB = _PA + HALF


def _sc_gather(table_packed, idx, addrbase):
    mesh = plsc.VectorSubcoreMesh(core_axis_name="c", subcore_axis_name="s")

    @functools.partial(
        pl.kernel,
        mesh=mesh,
        out_type=jax.ShapeDtypeStruct((KCH, BH * 4, HALF), jnp.int32),
        scratch_types=[
            pltpu.VMEM((ROWS_PER_W,), jnp.int32),
            pltpu.VMEM((16,), jnp.int32),
            pltpu.VMEM((CHUNK,), jnp.int32),
            pltpu.VMEM((CHUNK,), jnp.int32),
            pltpu.VMEM((CHUNK, HALF), jnp.int32),
            pltpu.VMEM((CHUNK, HALF), jnp.int32),
            pltpu.VMEM_SHARED((VOCAB, HALF), jnp.int32),
            pltpu.SemaphoreType.DMA,
            pltpu.SemaphoreType.DMA,
            pltpu.SemaphoreType.DMA,
            pltpu.SemaphoreType.DMA,
        ],
        compiler_params=pltpu.CompilerParams(use_tc_tiling_on_sc=False,
                                             needs_layout_passes=False),
    )
    def k(table_hbm, idx_hbm, ab_hbm, out_hbm, idx_v, addr_v, ib0, ib1,
          buf0, buf1, spm_tab, g0, g1, w0, w1):
        wid = lax.axis_index("s") * NC + lax.axis_index("c")

        @pl.when(lax.axis_index("s") == 0)
        def _():
            pltpu.sync_copy(table_hbm, spm_tab)

        pltpu.sync_copy(idx_hbm.at[pl.ds(wid * ROWS_PER_W, ROWS_PER_W)],
                        idx_v)
        pltpu.sync_copy(ab_hbm, addr_v)
        plsc.subcore_barrier()

        def build(ib):
            def bb(j, carry):
                a = addr_v[...]
                ib[pl.ds(j * 16, 16)] = plsc.load_gather(idx_v, [a])
                addr_v[...] = a + 4 * SEQ
                return carry

            lax.fori_loop(0, CHUNK // 16, bb, 0)
            addr_v[...] = addr_v[...] - (ROWS_PER_W - 4)

        def gat(buf, sem, ib):
            return pltpu.async_copy(spm_tab.at[ib], buf, sem)

        def wr(kk, buf, sem):
            off = pl.multiple_of(wid * CHUNK, CHUNK)
            return pltpu.async_copy(buf, out_hbm.at[kk, pl.ds(off, CHUNK)],
                                    sem)

        ibs = (ib0, ib1)
        bufs = (buf0, buf1)
        gsem = (g0, g1)
        wsem = (w0, w1)

        build(ibs[0])
        ghs = {0: gat(bufs[0], gsem[0], ibs[0])}
        whs = {}
        for t in range(KCH):
            if t + 1 < KCH:
                build(ibs[(t + 1) % 2])
                if t >= 1:
                    whs[t - 1].wait()
                ghs[t + 1] = gat(bufs[(t + 1) % 2], gsem[(t + 1) % 2],
                                 ibs[(t + 1) % 2])
            ghs[t].wait()
            whs[t] = wr(t, bufs[t % 2], wsem[t % 2])
        whs[KCH - 2].wait()
        whs[KCH - 1].wait()

    return k(table_packed, idx, addrbase)


TB = 2048


def _mlp_body(x_ref, w1_ref, b1_ref, w2_ref, b2_ref, w3_ref, b3_ref, o_ref):
    h = None
    for k in range(KCH):
        xi = x_ref[k]
        xa = lax.bitcast_convert_type(xi << 16, jnp.float32)
        xb = lax.bitcast_convert_type(xi & jnp.int32(-65536), jnp.float32)
        d = jnp.dot(xa.astype(jnp.bfloat16), w1_ref[k, :128],
                    preferred_element_type=jnp.float32)
        d += jnp.dot(xb.astype(jnp.bfloat16), w1_ref[k, 128:],
                     preferred_element_type=jnp.float32)
        h = d if h is None else h + d
    h = jnp.maximum(h + b1_ref[...], 0.0)
    h = jnp.dot(h, w2_ref[...], preferred_element_type=jnp.float32)
    h = jnp.maximum(h + b2_ref[...], 0.0)
    o = jnp.dot(h, w3_ref[...], preferred_element_type=jnp.float32)
    o_ref[...] = o + b3_ref[...]


def _mlp(x, W1ab, b1, W2, b2, W3, b3):
    grid = (BH // TB,)
    return pl.pallas_call(
        _mlp_body,
        grid=grid,
        in_specs=[
            pl.BlockSpec((KCH, TB, 128), lambda i: (0, i, 0)),
            pl.BlockSpec((KCH, 256, 128), lambda i: (0, 0, 0)),
            pl.BlockSpec((1, 128), lambda i: (0, 0)),
            pl.BlockSpec((128, 64), lambda i: (0, 0)),
            pl.BlockSpec((1, 64), lambda i: (0, 0)),
            pl.BlockSpec((64, 2), lambda i: (0, 0)),
            pl.BlockSpec((1, 2), lambda i: (0, 0)),
        ],
        out_specs=pl.BlockSpec((TB, 2), lambda i: (i, 0)),
        out_shape=jax.ShapeDtypeStruct((BH, 2), jnp.float32),
    )(x, W1ab, b1, W2, b2, W3, b3)


def kernel(inputs, table, W1, b1, W2, b2, W3, b3):
    tb = table.astype(jnp.bfloat16)
    lo = lax.bitcast_convert_type(tb[:, :HALF], jnp.uint16).astype(jnp.uint32)
    hi = lax.bitcast_convert_type(tb[:, HALF:], jnp.uint16).astype(jnp.uint32)
    packed = lax.bitcast_convert_type((hi << 16) | lo, jnp.int32)
    addrbase = jnp.asarray(
        [(l // 4) * SEQ + l % 4 for l in range(16)], dtype=jnp.int32)
    W1ab = jnp.concatenate([W1[_PA], W1[_PB]], axis=1)
    W1ab = W1ab.astype(jnp.bfloat16)
    b1r, b2r, b3r = b1.reshape(1, -1), b2.reshape(1, -1), b3.reshape(1, -1)

    outs = []
    for h in range(NHALVES):
        idx_h = inputs[h * BH:(h + 1) * BH].astype(jnp.int32).reshape(-1)
        x = _sc_gather(packed, idx_h, addrbase)
        x = x.reshape(KCH, BH, 128)
        outs.append(_mlp(x, W1ab, b1r, W2, b2r, W3, b3r))
    return jnp.concatenate(outs, axis=0)
